# Initial kernel scaffold; baseline (speedup 1.0000x reference)
import jax
import jax.numpy as jnp
from jax.experimental import pallas as pl


def kernel(x, perm, edge_index, W, att_src, att_dst, W_head, b_head):
    raise NotImplementedError("write your pallas kernel here")


# speedup vs baseline: 15.3969x; 15.3969x over previous
import jax
import jax.numpy as jnp
from jax import lax
from jax.experimental import pallas as pl
from jax.experimental.pallas import tpu as pltpu
from jax.experimental.pallas import tpu_sc as plsc

N = 10000
D = 128
E = 320000
NC = 2
NS = 16
NW = NC * NS
EPW = E // NW
CHUNK = 80
NCHUNK = EPW // CHUNK
RPT = 640
LASTR = N - (NS - 1) * RPT


def _k1_body(x_ref, w_ref, as_ref, ad_ref, hx_ref, ax_ref, adx_ref, m_ref,
             mx_sc):
    i = pl.program_id(0)
    hb = jnp.dot(x_ref[...], w_ref[...], preferred_element_type=jnp.float32)
    hx_ref[...] = hb
    axb = jnp.sum(hb * as_ref[...], axis=1, keepdims=True)
    adxb = jnp.sum(hb * ad_ref[...], axis=1, keepdims=True)
    ax_ref[...] = jnp.broadcast_to(axb, ax_ref.shape)
    adx_ref[...] = jnp.broadcast_to(adxb, adx_ref.shape)

    @pl.when(i == 0)
    def _():
        mx_sc[0, 0] = -jnp.inf
        mx_sc[0, 1] = -jnp.inf

    mx_sc[0, 0] = jnp.maximum(mx_sc[0, 0], jnp.max(axb))
    mx_sc[0, 1] = jnp.maximum(mx_sc[0, 1], jnp.max(adxb))

    @pl.when(i == pl.num_programs(0) - 1)
    def _():
        r = mx_sc[0, 0] + mx_sc[0, 1]
        m_ref[...] = jnp.where(r > 0.0, r, 0.2 * r).reshape(1, 1)


def _k1(x, W, att_src, att_dst):
    BN = 1000
    return pl.pallas_call(
        _k1_body,
        grid=(N // BN,),
        in_specs=[
            pl.BlockSpec((BN, D), lambda i: (i, 0)),
            pl.BlockSpec((D, D), lambda i: (0, 0)),
            pl.BlockSpec((1, D), lambda i: (0, 0)),
            pl.BlockSpec((1, D), lambda i: (0, 0)),
        ],
        out_specs=[
            pl.BlockSpec((BN, D), lambda i: (i, 0)),
            pl.BlockSpec((BN, 16), lambda i: (i, 0)),
            pl.BlockSpec((BN, 16), lambda i: (i, 0)),
            pl.BlockSpec((1, 1), lambda i: (0, 0)),
        ],
        out_shape=[
            jax.ShapeDtypeStruct((N, D), jnp.float32),
            jax.ShapeDtypeStruct((N, 16), jnp.float32),
            jax.ShapeDtypeStruct((N, 16), jnp.float32),
            jax.ShapeDtypeStruct((1, 1), jnp.float32),
        ],
        scratch_shapes=[pltpu.SMEM((1, 2), jnp.float32)],
    )(x, W, att_src.reshape(1, D), att_dst.reshape(1, D))


def _k2_body(hx_h, ax_h, adx_h, perm_h, src_h, dst_h, m_h,
             acc_h, den_h, hxp0_h, hxp1_h, axp0_h, axp1_h, adxp0_h, adxp1_h,
             src_v, dst_v, p_v, p_mat, rows_v, avb_v, bvb_v, m_v,
             sem, out_sh, den_sh):
    cid = lax.axis_index("c")
    sid = lax.axis_index("s")
    wid = sid * NC + cid
    nk = jnp.where(sid == NS - 1, LASTR // CHUNK, RPT // CHUNK)

    pltpu.sync_copy(m_h, m_v)

    zero16 = jnp.zeros((16,), jnp.float32)

    def _zero_rows(r, c):
        for j in range(D // 16):
            rows_v[r, pl.ds(j * 16, 16)] = zero16
        return c
    lax.fori_loop(0, CHUNK, _zero_rows, 0)

    def _zero_pm(r, c):
        p_mat[r, pl.ds(0, 16)] = zero16
        return c
    lax.fori_loop(0, CHUNK, _zero_pm, 0)

    def _zero_stripes(k, c):
        off = pl.ds(sid * RPT + k * CHUNK, CHUNK)
        pltpu.sync_copy(rows_v, out_sh.at[off])
        pltpu.sync_copy(p_mat, den_sh.at[off])
        return c
    lax.fori_loop(0, nk, _zero_stripes, 0)

    def _build(k, c):
        off = sid * RPT + k * CHUNK
        pltpu.sync_copy(perm_h.at[pl.ds(off, CHUNK)], src_v)
        pltpu.async_copy(hx_h.at[src_v], rows_v, sem).wait()
        pltpu.async_copy(ax_h.at[src_v], avb_v, sem).wait()
        pltpu.async_copy(adx_h.at[src_v], bvb_v, sem).wait()

        @pl.when(cid == 0)
        def _():
            pltpu.sync_copy(rows_v, hxp0_h.at[pl.ds(off, CHUNK)])
            pltpu.sync_copy(avb_v, axp0_h.at[pl.ds(off, CHUNK)])
            pltpu.sync_copy(bvb_v, adxp0_h.at[pl.ds(off, CHUNK)])

        @pl.when(cid == 1)
        def _():
            pltpu.sync_copy(rows_v, hxp1_h.at[pl.ds(off, CHUNK)])
            pltpu.sync_copy(avb_v, axp1_h.at[pl.ds(off, CHUNK)])
            pltpu.sync_copy(bvb_v, adxp1_h.at[pl.ds(off, CHUNK)])
        return c
    lax.fori_loop(0, nk, _build, 0)

    plsc.subcore_barrier()

    mv = m_v[...]
    ziota = lax.iota(jnp.int32, 16) * 0

    def _chunk(i, c):
        base = wid * EPW + i * CHUNK
        pltpu.sync_copy(src_h.at[pl.ds(base, CHUNK)], src_v)
        pltpu.sync_copy(dst_h.at[pl.ds(base, CHUNK)], dst_v)

        @pl.when(cid == 0)
        def _():
            pltpu.async_copy(axp0_h.at[src_v], avb_v, sem).wait()
            pltpu.async_copy(adxp0_h.at[dst_v], bvb_v, sem).wait()
            pltpu.async_copy(hxp0_h.at[src_v], rows_v, sem).wait()

        @pl.when(cid == 1)
        def _():
            pltpu.async_copy(axp1_h.at[src_v], avb_v, sem).wait()
            pltpu.async_copy(adxp1_h.at[dst_v], bvb_v, sem).wait()
            pltpu.async_copy(hxp1_h.at[src_v], rows_v, sem).wait()

        for j in range(CHUNK // 16):
            sl = pl.ds(j * 16, 16)
            ridx = lax.iota(jnp.int32, 16) + j * 16
            av = plsc.load_gather(avb_v, [ridx, ziota])
            bv = plsc.load_gather(bvb_v, [ridx, ziota])
            raw = av + bv
            e = jnp.where(raw > 0.0, raw, raw * 0.2)
            p_v[sl] = jnp.exp(e - mv)

        def _scale(r, cc):
            pr = plsc.load_gather(p_v, [jnp.zeros((16,), jnp.int32) + r])
            for j in range(D // 16):
                csl = pl.ds(j * 16, 16)
                rows_v[r, csl] = rows_v[r, csl] * pr
            p_mat[r, pl.ds(0, 16)] = pr
            return cc
        lax.fori_loop(0, CHUNK, _scale, 0)

        pltpu.sync_copy(rows_v, out_sh.at[dst_v], add=True)
        pltpu.sync_copy(p_mat, den_sh.at[dst_v], add=True)
        return c

    lax.fori_loop(0, NCHUNK, _chunk, 0)

    plsc.subcore_barrier()

    @pl.when(sid < NS - 1)
    def _():
        pltpu.sync_copy(out_sh.at[pl.ds(sid * RPT, RPT)],
                        acc_h.at[pl.ds(cid * N + sid * RPT, RPT)])
        pltpu.sync_copy(den_sh.at[pl.ds(sid * RPT, RPT)],
                        den_h.at[pl.ds(cid * N + sid * RPT, RPT)])

    @pl.when(sid == NS - 1)
    def _():
        pltpu.sync_copy(out_sh.at[pl.ds((NS - 1) * RPT, LASTR)],
                        acc_h.at[pl.ds(cid * N + (NS - 1) * RPT, LASTR)])
        pltpu.sync_copy(den_sh.at[pl.ds((NS - 1) * RPT, LASTR)],
                        den_h.at[pl.ds(cid * N + (NS - 1) * RPT, LASTR)])


def _k2(hx, ax16, adx16, perm_i, src, dst, mvec):
    mesh = plsc.VectorSubcoreMesh(core_axis_name="c", subcore_axis_name="s")
    f = pl.kernel(
        _k2_body,
        out_type=[
            jax.ShapeDtypeStruct((NC * N, D), jnp.float32),
            jax.ShapeDtypeStruct((NC * N, 16), jnp.float32),
            jax.ShapeDtypeStruct((N, D), jnp.float32),
            jax.ShapeDtypeStruct((N, D), jnp.float32),
            jax.ShapeDtypeStruct((N, 16), jnp.float32),
            jax.ShapeDtypeStruct((N, 16), jnp.float32),
            jax.ShapeDtypeStruct((N, 16), jnp.float32),
            jax.ShapeDtypeStruct((N, 16), jnp.float32),
        ],
        mesh=mesh,
        compiler_params=pltpu.CompilerParams(needs_layout_passes=False,
                                            use_tc_tiling_on_sc=False),
        scratch_types=[
            pltpu.VMEM((CHUNK,), jnp.int32),
            pltpu.VMEM((CHUNK,), jnp.int32),
            pltpu.VMEM((CHUNK,), jnp.float32),
            pltpu.VMEM((CHUNK, 16), jnp.float32),
            pltpu.VMEM((CHUNK, D), jnp.float32),
            pltpu.VMEM((CHUNK, 16), jnp.float32),
            pltpu.VMEM((CHUNK, 16), jnp.float32),
            pltpu.VMEM((16,), jnp.float32),
            pltpu.SemaphoreType.DMA,
            pltpu.VMEM_SHARED((N, D), jnp.float32),
            pltpu.VMEM_SHARED((N, 16), jnp.float32),
        ],
    )
    return f(hx, ax16, adx16, perm_i, src, dst, mvec)


def _k3_body(a0_ref, a1_ref, d0_ref, d1_ref, wh_ref, bh_ref, out_ref,
             acc_sc):
    i = pl.program_id(0)
    dsum = (d0_ref[...] + d1_ref[...])[:, 0]
    a = a0_ref[...] + a1_ref[...]
    o = a / (dsum[:, None] + 1e-16)
    o = jnp.where(o > 0.0, o, jnp.exp(jnp.minimum(o, 0.0)) - 1.0)

    @pl.when(i == 0)
    def _():
        acc_sc[...] = jnp.zeros_like(acc_sc)

    acc_sc[...] += jnp.sum(o, axis=0, keepdims=True)

    @pl.when(i == pl.num_programs(0) - 1)
    def _():
        out_ref[...] = (jnp.sum(acc_sc[...] * wh_ref[...].T) / N
                        + jnp.sum(bh_ref[...])).reshape(1, 1)


def _k3(acc0, acc1, den0, den1, W_head, b_head):
    BN = 1000
    return pl.pallas_call(
        _k3_body,
        grid=(N // BN,),
        in_specs=[
            pl.BlockSpec((BN, D), lambda i: (i, 0)),
            pl.BlockSpec((BN, D), lambda i: (i, 0)),
            pl.BlockSpec((BN, 16), lambda i: (i, 0)),
            pl.BlockSpec((BN, 16), lambda i: (i, 0)),
            pl.BlockSpec((D, 1), lambda i: (0, 0)),
            pl.BlockSpec((1, 1), lambda i: (0, 0)),
        ],
        out_specs=pl.BlockSpec((1, 1), lambda i: (0, 0)),
        out_shape=jax.ShapeDtypeStruct((1, 1), jnp.float32),
        scratch_shapes=[pltpu.VMEM((1, D), jnp.float32)],
    )(acc0, acc1, den0, den1, W_head, b_head.reshape(1, 1))


def kernel(x, perm, edge_index, W, att_src, att_dst, W_head, b_head):
    hx, ax16, adx16, M = _k1(x, W, att_src, att_dst)

    perm_i = perm.astype(jnp.int32)
    src = edge_index[0].astype(jnp.int32)
    dst = edge_index[1].astype(jnp.int32)
    mvec = jnp.broadcast_to(M.reshape(1), (16,))

    acc, den = _k2(hx, ax16, adx16, perm_i, src, dst, mvec)[:2]

    pred = _k3(acc[:N], acc[N:], den[:N], den[N:], W_head, b_head)
    return pred.reshape(1)


# speedup vs baseline: 21.6448x; 1.4058x over previous
import jax
import jax.numpy as jnp
from jax import lax
from jax.experimental import pallas as pl
from jax.experimental.pallas import tpu as pltpu
from jax.experimental.pallas import tpu_sc as plsc

N = 10000
D = 128
E = 320000
NC = 2
NS = 16
NW = NC * NS
EPW = E // NW
CHUNK = 80
NCHUNK = EPW // CHUNK
RPT = 640
LASTR = N - (NS - 1) * RPT


def _k1_body(x_ref, w_ref, as_ref, ad_ref, hx_ref, ax_ref, adx_ref, m_ref,
             mx_sc):
    i = pl.program_id(0)
    hb = jnp.dot(x_ref[...], w_ref[...], preferred_element_type=jnp.float32)
    hx_ref[...] = hb
    axb = jnp.sum(hb * as_ref[...], axis=1, keepdims=True)
    adxb = jnp.sum(hb * ad_ref[...], axis=1, keepdims=True)
    ax_ref[...] = jnp.broadcast_to(axb, ax_ref.shape)
    adx_ref[...] = jnp.broadcast_to(adxb, adx_ref.shape)

    @pl.when(i == 0)
    def _():
        mx_sc[0, 0] = -jnp.inf
        mx_sc[0, 1] = -jnp.inf

    mx_sc[0, 0] = jnp.maximum(mx_sc[0, 0], jnp.max(axb))
    mx_sc[0, 1] = jnp.maximum(mx_sc[0, 1], jnp.max(adxb))

    @pl.when(i == pl.num_programs(0) - 1)
    def _():
        r = mx_sc[0, 0] + mx_sc[0, 1]
        m_ref[...] = jnp.where(r > 0.0, r, 0.2 * r).reshape(1, 1)


def _k1(x, W, att_src, att_dst):
    BN = 1000
    return pl.pallas_call(
        _k1_body,
        grid=(N // BN,),
        in_specs=[
            pl.BlockSpec((BN, D), lambda i: (i, 0)),
            pl.BlockSpec((D, D), lambda i: (0, 0)),
            pl.BlockSpec((1, D), lambda i: (0, 0)),
            pl.BlockSpec((1, D), lambda i: (0, 0)),
        ],
        out_specs=[
            pl.BlockSpec((BN, D), lambda i: (i, 0)),
            pl.BlockSpec((BN, 16), lambda i: (i, 0)),
            pl.BlockSpec((BN, 16), lambda i: (i, 0)),
            pl.BlockSpec((1, 1), lambda i: (0, 0)),
        ],
        out_shape=[
            jax.ShapeDtypeStruct((N, D), jnp.float32),
            jax.ShapeDtypeStruct((N, 16), jnp.float32),
            jax.ShapeDtypeStruct((N, 16), jnp.float32),
            jax.ShapeDtypeStruct((1, 1), jnp.float32),
        ],
        scratch_shapes=[pltpu.SMEM((1, 2), jnp.float32)],
    )(x, W, att_src.reshape(1, D), att_dst.reshape(1, D))


def _k2_body(hx_h, ax_h, adx_h, perm_h, src_h, dst_h, m_h,
             acc_h, den_h, hxp0_h, hxp1_h, axp0_h, axp1_h, adxp0_h, adxp1_h,
             src_v, dst_v, p_v, p_mat, rows_v, avb_v, bvb_v, m_v,
             sem, out_sh, den_sh):
    cid = lax.axis_index("c")
    sid = lax.axis_index("s")
    wid = sid * NC + cid
    nk = jnp.where(sid == NS - 1, LASTR // CHUNK, RPT // CHUNK)

    pltpu.sync_copy(m_h, m_v)

    zero16 = jnp.zeros((16,), jnp.float32)

    def _zero_rows(r, c):
        for j in range(D // 16):
            rows_v[r, pl.ds(j * 16, 16)] = zero16
        return c
    lax.fori_loop(0, CHUNK, _zero_rows, 0)

    def _zero_pm(r, c):
        p_mat[r, pl.ds(0, 16)] = zero16
        return c
    lax.fori_loop(0, CHUNK, _zero_pm, 0)

    def _zero_stripes(k, c):
        off = pl.ds(sid * RPT + k * CHUNK, CHUNK)
        pltpu.sync_copy(rows_v, out_sh.at[off])
        pltpu.sync_copy(p_mat, den_sh.at[off])
        return c
    lax.fori_loop(0, nk, _zero_stripes, 0)

    def _build(k, c):
        off = sid * RPT + k * CHUNK
        pltpu.sync_copy(perm_h.at[pl.ds(off, CHUNK)], src_v)
        g1 = pltpu.async_copy(hx_h.at[src_v], rows_v, sem)
        g2 = pltpu.async_copy(ax_h.at[src_v], avb_v, sem)
        g3 = pltpu.async_copy(adx_h.at[src_v], bvb_v, sem)
        g1.wait()
        g2.wait()
        g3.wait()

        @pl.when(cid == 0)
        def _():
            pltpu.sync_copy(rows_v, hxp0_h.at[pl.ds(off, CHUNK)])
            pltpu.sync_copy(avb_v, axp0_h.at[pl.ds(off, CHUNK)])
            pltpu.sync_copy(bvb_v, adxp0_h.at[pl.ds(off, CHUNK)])

        @pl.when(cid == 1)
        def _():
            pltpu.sync_copy(rows_v, hxp1_h.at[pl.ds(off, CHUNK)])
            pltpu.sync_copy(avb_v, axp1_h.at[pl.ds(off, CHUNK)])
            pltpu.sync_copy(bvb_v, adxp1_h.at[pl.ds(off, CHUNK)])
        return c
    lax.fori_loop(0, nk, _build, 0)

    plsc.subcore_barrier()

    mv = m_v[...]
    ziota = lax.iota(jnp.int32, 16) * 0

    def _chunk(i, c):
        base = wid * EPW + i * CHUNK
        ci = pltpu.async_copy(src_h.at[pl.ds(base, CHUNK)], src_v, sem)
        cj = pltpu.async_copy(dst_h.at[pl.ds(base, CHUNK)], dst_v, sem)
        ci.wait()
        cj.wait()

        @pl.when(cid == 0)
        def _():
            g1 = pltpu.async_copy(axp0_h.at[src_v], avb_v, sem)
            g2 = pltpu.async_copy(adxp0_h.at[dst_v], bvb_v, sem)
            g3 = pltpu.async_copy(hxp0_h.at[src_v], rows_v, sem)
            g1.wait()
            g2.wait()
            g3.wait()

        @pl.when(cid == 1)
        def _():
            g1 = pltpu.async_copy(axp1_h.at[src_v], avb_v, sem)
            g2 = pltpu.async_copy(adxp1_h.at[dst_v], bvb_v, sem)
            g3 = pltpu.async_copy(hxp1_h.at[src_v], rows_v, sem)
            g1.wait()
            g2.wait()
            g3.wait()

        for j in range(CHUNK // 16):
            sl = pl.ds(j * 16, 16)
            ridx = lax.iota(jnp.int32, 16) + j * 16
            av = plsc.load_gather(avb_v, [ridx, ziota])
            bv = plsc.load_gather(bvb_v, [ridx, ziota])
            raw = av + bv
            e = jnp.where(raw > 0.0, raw, raw * 0.2)
            p_v[sl] = jnp.exp(e - mv)

        def _scale(r, cc):
            pr = plsc.load_gather(p_v, [jnp.zeros((16,), jnp.int32) + r])
            for j in range(D // 16):
                csl = pl.ds(j * 16, 16)
                rows_v[r, csl] = rows_v[r, csl] * pr
            p_mat[r, pl.ds(0, 16)] = pr
            return cc
        lax.fori_loop(0, CHUNK, _scale, 0)

        s1 = pltpu.async_copy(rows_v, out_sh.at[dst_v], sem, add=True)
        s2 = pltpu.async_copy(p_mat, den_sh.at[dst_v], sem, add=True)
        s1.wait()
        s2.wait()
        return c

    lax.fori_loop(0, NCHUNK, _chunk, 0)

    plsc.subcore_barrier()

    @pl.when(sid < NS - 1)
    def _():
        pltpu.sync_copy(out_sh.at[pl.ds(sid * RPT, RPT)],
                        acc_h.at[pl.ds(cid * N + sid * RPT, RPT)])
        pltpu.sync_copy(den_sh.at[pl.ds(sid * RPT, RPT)],
                        den_h.at[pl.ds(cid * N + sid * RPT, RPT)])

    @pl.when(sid == NS - 1)
    def _():
        pltpu.sync_copy(out_sh.at[pl.ds((NS - 1) * RPT, LASTR)],
                        acc_h.at[pl.ds(cid * N + (NS - 1) * RPT, LASTR)])
        pltpu.sync_copy(den_sh.at[pl.ds((NS - 1) * RPT, LASTR)],
                        den_h.at[pl.ds(cid * N + (NS - 1) * RPT, LASTR)])


def _k2(hx, ax16, adx16, perm_i, src, dst, mvec):
    mesh = plsc.VectorSubcoreMesh(core_axis_name="c", subcore_axis_name="s")
    f = pl.kernel(
        _k2_body,
        out_type=[
            jax.ShapeDtypeStruct((NC * N, D), jnp.float32),
            jax.ShapeDtypeStruct((NC * N, 16), jnp.float32),
            jax.ShapeDtypeStruct((N, D), jnp.float32),
            jax.ShapeDtypeStruct((N, D), jnp.float32),
            jax.ShapeDtypeStruct((N, 16), jnp.float32),
            jax.ShapeDtypeStruct((N, 16), jnp.float32),
            jax.ShapeDtypeStruct((N, 16), jnp.float32),
            jax.ShapeDtypeStruct((N, 16), jnp.float32),
        ],
        mesh=mesh,
        compiler_params=pltpu.CompilerParams(needs_layout_passes=False,
                                            use_tc_tiling_on_sc=False),
        scratch_types=[
            pltpu.VMEM((CHUNK,), jnp.int32),
            pltpu.VMEM((CHUNK,), jnp.int32),
            pltpu.VMEM((CHUNK,), jnp.float32),
            pltpu.VMEM((CHUNK, 16), jnp.float32),
            pltpu.VMEM((CHUNK, D), jnp.float32),
            pltpu.VMEM((CHUNK, 16), jnp.float32),
            pltpu.VMEM((CHUNK, 16), jnp.float32),
            pltpu.VMEM((16,), jnp.float32),
            pltpu.SemaphoreType.DMA,
            pltpu.VMEM_SHARED((N, D), jnp.float32),
            pltpu.VMEM_SHARED((N, 16), jnp.float32),
        ],
    )
    return f(hx, ax16, adx16, perm_i, src, dst, mvec)


def _k3_body(a0_ref, a1_ref, d0_ref, d1_ref, wh_ref, bh_ref, out_ref,
             acc_sc):
    i = pl.program_id(0)
    dsum = (d0_ref[...] + d1_ref[...])[:, 0]
    a = a0_ref[...] + a1_ref[...]
    o = a / (dsum[:, None] + 1e-16)
    o = jnp.where(o > 0.0, o, jnp.exp(jnp.minimum(o, 0.0)) - 1.0)

    @pl.when(i == 0)
    def _():
        acc_sc[...] = jnp.zeros_like(acc_sc)

    acc_sc[...] += jnp.sum(o, axis=0, keepdims=True)

    @pl.when(i == pl.num_programs(0) - 1)
    def _():
        out_ref[...] = (jnp.sum(acc_sc[...] * wh_ref[...].T) / N
                        + jnp.sum(bh_ref[...])).reshape(1, 1)


def _k3(acc0, acc1, den0, den1, W_head, b_head):
    BN = 1000
    return pl.pallas_call(
        _k3_body,
        grid=(N // BN,),
        in_specs=[
            pl.BlockSpec((BN, D), lambda i: (i, 0)),
            pl.BlockSpec((BN, D), lambda i: (i, 0)),
            pl.BlockSpec((BN, 16), lambda i: (i, 0)),
            pl.BlockSpec((BN, 16), lambda i: (i, 0)),
            pl.BlockSpec((D, 1), lambda i: (0, 0)),
            pl.BlockSpec((1, 1), lambda i: (0, 0)),
        ],
        out_specs=pl.BlockSpec((1, 1), lambda i: (0, 0)),
        out_shape=jax.ShapeDtypeStruct((1, 1), jnp.float32),
        scratch_shapes=[pltpu.VMEM((1, D), jnp.float32)],
    )(acc0, acc1, den0, den1, W_head, b_head.reshape(1, 1))


def kernel(x, perm, edge_index, W, att_src, att_dst, W_head, b_head):
    hx, ax16, adx16, M = _k1(x, W, att_src, att_dst)

    perm_i = perm.astype(jnp.int32)
    src = edge_index[0].astype(jnp.int32)
    dst = edge_index[1].astype(jnp.int32)
    mvec = jnp.broadcast_to(M.reshape(1), (16,))

    acc, den = _k2(hx, ax16, adx16, perm_i, src, dst, mvec)[:2]

    pred = _k3(acc[:N], acc[N:], den[:N], den[N:], W_head, b_head)
    return pred.reshape(1)


# speedup vs baseline: 24.0446x; 1.1109x over previous
import jax
import jax.numpy as jnp
from jax import lax
from jax.experimental import pallas as pl
from jax.experimental.pallas import tpu as pltpu
from jax.experimental.pallas import tpu_sc as plsc

N = 10000
D = 128
E = 320000
NC = 2
NS = 16
NW = NC * NS
EPW = E // NW
CHUNK = 80
NCHUNK = EPW // CHUNK
RPT = 640
LASTR = N - (NS - 1) * RPT


def _k1_body(x_ref, w_ref, as_ref, ad_ref, hx_ref, ax_ref, adx_ref, m_ref,
             mx_sc):
    i = pl.program_id(0)
    hb = jnp.dot(x_ref[...], w_ref[...], preferred_element_type=jnp.float32)
    hx_ref[...] = hb
    axb = jnp.sum(hb * as_ref[...], axis=1, keepdims=True)
    adxb = jnp.sum(hb * ad_ref[...], axis=1, keepdims=True)
    ax_ref[...] = jnp.broadcast_to(axb, ax_ref.shape)
    adx_ref[...] = jnp.broadcast_to(adxb, adx_ref.shape)

    @pl.when(i == 0)
    def _():
        mx_sc[0, 0] = -jnp.inf
        mx_sc[0, 1] = -jnp.inf

    mx_sc[0, 0] = jnp.maximum(mx_sc[0, 0], jnp.max(axb))
    mx_sc[0, 1] = jnp.maximum(mx_sc[0, 1], jnp.max(adxb))

    @pl.when(i == pl.num_programs(0) - 1)
    def _():
        r = mx_sc[0, 0] + mx_sc[0, 1]
        m_ref[...] = jnp.where(r > 0.0, r, 0.2 * r).reshape(1, 1)


def _k1(x, W, att_src, att_dst):
    BN = 1000
    return pl.pallas_call(
        _k1_body,
        grid=(N // BN,),
        in_specs=[
            pl.BlockSpec((BN, D), lambda i: (i, 0)),
            pl.BlockSpec((D, D), lambda i: (0, 0)),
            pl.BlockSpec((1, D), lambda i: (0, 0)),
            pl.BlockSpec((1, D), lambda i: (0, 0)),
        ],
        out_specs=[
            pl.BlockSpec((BN, D), lambda i: (i, 0)),
            pl.BlockSpec((BN, 16), lambda i: (i, 0)),
            pl.BlockSpec((BN, 16), lambda i: (i, 0)),
            pl.BlockSpec((1, 1), lambda i: (0, 0)),
        ],
        out_shape=[
            jax.ShapeDtypeStruct((N, D), jnp.float32),
            jax.ShapeDtypeStruct((N, 16), jnp.float32),
            jax.ShapeDtypeStruct((N, 16), jnp.float32),
            jax.ShapeDtypeStruct((1, 1), jnp.float32),
        ],
        scratch_shapes=[pltpu.SMEM((1, 2), jnp.float32)],
    )(x, W, att_src.reshape(1, D), att_dst.reshape(1, D))


def _k2_body(hx_h, ax_h, adx_h, perm_h, src_h, dst_h, m_h,
             acc_h, den_h, hxp0_h, hxp1_h, axp0_h, axp1_h, adxp0_h, adxp1_h,
             src_v, dsts_v, p_v, p_mat, rows_v, avb_v, bvb_v, m_v,
             src_all, dst_all, sem, out_sh, den_sh):
    cid = lax.axis_index("c")
    sid = lax.axis_index("s")
    wid = sid * NC + cid
    nk = jnp.where(sid == NS - 1, LASTR // CHUNK, RPT // CHUNK)

    pltpu.sync_copy(m_h, m_v)

    zero16 = jnp.zeros((16,), jnp.float32)

    def _zero_rows(r, c):
        for j in range(D // 16):
            rows_v[r, pl.ds(j * 16, 16)] = zero16
        return c
    lax.fori_loop(0, CHUNK, _zero_rows, 0)

    def _zero_pm(r, c):
        p_mat[r, pl.ds(0, 16)] = zero16
        return c
    lax.fori_loop(0, CHUNK, _zero_pm, 0)

    def _zero_stripes(k, c):
        off = pl.ds(sid * RPT + k * CHUNK, CHUNK)
        pltpu.sync_copy(rows_v, out_sh.at[off])
        pltpu.sync_copy(p_mat, den_sh.at[off])
        return c
    lax.fori_loop(0, nk, _zero_stripes, 0)

    def _build(k, c):
        off = sid * RPT + k * CHUNK
        pltpu.sync_copy(perm_h.at[pl.ds(off, CHUNK)], src_v)
        g1 = pltpu.async_copy(hx_h.at[src_v], rows_v, sem)
        g2 = pltpu.async_copy(ax_h.at[src_v], avb_v, sem)
        g3 = pltpu.async_copy(adx_h.at[src_v], bvb_v, sem)
        g1.wait()
        g2.wait()
        g3.wait()

        @pl.when(cid == 0)
        def _():
            pltpu.sync_copy(rows_v, hxp0_h.at[pl.ds(off, CHUNK)])
            pltpu.sync_copy(avb_v, axp0_h.at[pl.ds(off, CHUNK)])
            pltpu.sync_copy(bvb_v, adxp0_h.at[pl.ds(off, CHUNK)])

        @pl.when(cid == 1)
        def _():
            pltpu.sync_copy(rows_v, hxp1_h.at[pl.ds(off, CHUNK)])
            pltpu.sync_copy(avb_v, axp1_h.at[pl.ds(off, CHUNK)])
            pltpu.sync_copy(bvb_v, adxp1_h.at[pl.ds(off, CHUNK)])
        return c
    lax.fori_loop(0, nk, _build, 0)

    ca = pltpu.async_copy(src_h.at[pl.ds(wid * EPW, EPW)], src_all, sem)
    cb = pltpu.async_copy(dst_h.at[pl.ds(wid * EPW, EPW)], dst_all, sem)
    ca.wait()
    cb.wait()

    plsc.subcore_barrier()

    mv = m_v[...]
    ziota = lax.iota(jnp.int32, 16) * 0

    def _chunk(i, c):
        sl_s = src_all.at[pl.ds(i * CHUNK, CHUNK)]
        sl_d = dst_all.at[pl.ds(i * CHUNK, CHUNK)]
        cd = pltpu.async_copy(dst_h.at[pl.ds(wid * EPW + i * CHUNK, CHUNK)],
                              dsts_v, sem)

        @pl.when(cid == 0)
        def _():
            g1 = pltpu.async_copy(axp0_h.at[sl_s], avb_v, sem)
            g2 = pltpu.async_copy(adxp0_h.at[sl_d], bvb_v, sem)
            g3 = pltpu.async_copy(hxp0_h.at[sl_s], rows_v, sem)
            g1.wait()
            g2.wait()
            g3.wait()

        @pl.when(cid == 1)
        def _():
            g1 = pltpu.async_copy(axp1_h.at[sl_s], avb_v, sem)
            g2 = pltpu.async_copy(adxp1_h.at[sl_d], bvb_v, sem)
            g3 = pltpu.async_copy(hxp1_h.at[sl_s], rows_v, sem)
            g1.wait()
            g2.wait()
            g3.wait()
        cd.wait()

        for j in range(CHUNK // 16):
            sl = pl.ds(j * 16, 16)
            ridx = lax.iota(jnp.int32, 16) + j * 16
            av = plsc.load_gather(avb_v, [ridx, ziota])
            bv = plsc.load_gather(bvb_v, [ridx, ziota])
            raw = av + bv
            e = jnp.where(raw > 0.0, raw, raw * 0.2)
            p_v[sl] = jnp.exp(e - mv)

        def _scale(r, cc):
            pr = plsc.load_gather(p_v, [jnp.zeros((16,), jnp.int32) + r])
            for j in range(D // 16):
                csl = pl.ds(j * 16, 16)
                rows_v[r, csl] = rows_v[r, csl] * pr
            p_mat[r, pl.ds(0, 16)] = pr
            return cc
        lax.fori_loop(0, CHUNK, _scale, 0)

        s1 = pltpu.async_copy(rows_v, out_sh.at[dsts_v], sem, add=True)
        s2 = pltpu.async_copy(p_mat, den_sh.at[dsts_v], sem, add=True)
        s1.wait()
        s2.wait()
        return c

    lax.fori_loop(0, NCHUNK, _chunk, 0)

    plsc.subcore_barrier()

    @pl.when(sid < NS - 1)
    def _():
        pltpu.sync_copy(out_sh.at[pl.ds(sid * RPT, RPT)],
                        acc_h.at[pl.ds(cid * N + sid * RPT, RPT)])
        pltpu.sync_copy(den_sh.at[pl.ds(sid * RPT, RPT)],
                        den_h.at[pl.ds(cid * N + sid * RPT, RPT)])

    @pl.when(sid == NS - 1)
    def _():
        pltpu.sync_copy(out_sh.at[pl.ds((NS - 1) * RPT, LASTR)],
                        acc_h.at[pl.ds(cid * N + (NS - 1) * RPT, LASTR)])
        pltpu.sync_copy(den_sh.at[pl.ds((NS - 1) * RPT, LASTR)],
                        den_h.at[pl.ds(cid * N + (NS - 1) * RPT, LASTR)])


def _k2(hx, ax16, adx16, perm_i, src, dst, mvec):
    mesh = plsc.VectorSubcoreMesh(core_axis_name="c", subcore_axis_name="s")
    f = pl.kernel(
        _k2_body,
        out_type=[
            jax.ShapeDtypeStruct((NC * N, D), jnp.float32),
            jax.ShapeDtypeStruct((NC * N, 16), jnp.float32),
            jax.ShapeDtypeStruct((N, D), jnp.float32),
            jax.ShapeDtypeStruct((N, D), jnp.float32),
            jax.ShapeDtypeStruct((N, 16), jnp.float32),
            jax.ShapeDtypeStruct((N, 16), jnp.float32),
            jax.ShapeDtypeStruct((N, 16), jnp.float32),
            jax.ShapeDtypeStruct((N, 16), jnp.float32),
        ],
        mesh=mesh,
        compiler_params=pltpu.CompilerParams(needs_layout_passes=False,
                                            use_tc_tiling_on_sc=False),
        scratch_types=[
            pltpu.VMEM((CHUNK,), jnp.int32),
            pltpu.VMEM((CHUNK,), jnp.int32),
            pltpu.VMEM((CHUNK,), jnp.float32),
            pltpu.VMEM((CHUNK, 16), jnp.float32),
            pltpu.VMEM((CHUNK, D), jnp.float32),
            pltpu.VMEM((CHUNK, 16), jnp.float32),
            pltpu.VMEM((CHUNK, 16), jnp.float32),
            pltpu.VMEM((16,), jnp.float32),
            pltpu.VMEM((EPW,), jnp.int32),
            pltpu.VMEM((EPW,), jnp.int32),
            pltpu.SemaphoreType.DMA,
            pltpu.VMEM_SHARED((N, D), jnp.float32),
            pltpu.VMEM_SHARED((N, 16), jnp.float32),
        ],
    )
    return f(hx, ax16, adx16, perm_i, src, dst, mvec)


def _k3_body(a0_ref, a1_ref, d0_ref, d1_ref, wh_ref, bh_ref, out_ref,
             acc_sc):
    i = pl.program_id(0)
    dsum = (d0_ref[...] + d1_ref[...])[:, 0]
    a = a0_ref[...] + a1_ref[...]
    o = a / (dsum[:, None] + 1e-16)
    o = jnp.where(o > 0.0, o, jnp.exp(jnp.minimum(o, 0.0)) - 1.0)

    @pl.when(i == 0)
    def _():
        acc_sc[...] = jnp.zeros_like(acc_sc)

    acc_sc[...] += jnp.sum(o, axis=0, keepdims=True)

    @pl.when(i == pl.num_programs(0) - 1)
    def _():
        out_ref[...] = (jnp.sum(acc_sc[...] * wh_ref[...].T) / N
                        + jnp.sum(bh_ref[...])).reshape(1, 1)


def _k3(acc0, acc1, den0, den1, W_head, b_head):
    BN = 1000
    return pl.pallas_call(
        _k3_body,
        grid=(N // BN,),
        in_specs=[
            pl.BlockSpec((BN, D), lambda i: (i, 0)),
            pl.BlockSpec((BN, D), lambda i: (i, 0)),
            pl.BlockSpec((BN, 16), lambda i: (i, 0)),
            pl.BlockSpec((BN, 16), lambda i: (i, 0)),
            pl.BlockSpec((D, 1), lambda i: (0, 0)),
            pl.BlockSpec((1, 1), lambda i: (0, 0)),
        ],
        out_specs=pl.BlockSpec((1, 1), lambda i: (0, 0)),
        out_shape=jax.ShapeDtypeStruct((1, 1), jnp.float32),
        scratch_shapes=[pltpu.VMEM((1, D), jnp.float32)],
    )(acc0, acc1, den0, den1, W_head, b_head.reshape(1, 1))


def kernel(x, perm, edge_index, W, att_src, att_dst, W_head, b_head):
    hx, ax16, adx16, M = _k1(x, W, att_src, att_dst)

    perm_i = perm.astype(jnp.int32)
    src = edge_index[0].astype(jnp.int32)
    dst = edge_index[1].astype(jnp.int32)
    mvec = jnp.broadcast_to(M.reshape(1), (16,))

    acc, den = _k2(hx, ax16, adx16, perm_i, src, dst, mvec)[:2]

    pred = _k3(acc[:N], acc[N:], den[:N], den[N:], W_head, b_head)
    return pred.reshape(1)


# speedup vs baseline: 29.5237x; 1.2279x over previous
import jax
import jax.numpy as jnp
from jax import lax
from jax.experimental import pallas as pl
from jax.experimental.pallas import tpu as pltpu
from jax.experimental.pallas import tpu_sc as plsc

N = 10000
D = 128
E = 320000
NC = 2
NS = 16
NW = NC * NS
EPW = E // NW
CHUNK = 40
NCHUNK = EPW // CHUNK
RPT = 640
LASTR = N - (NS - 1) * RPT


def _k1_body(x_ref, w_ref, as_ref, ad_ref, hx_ref, ax_ref, adx_ref, m_ref,
             mx_sc):
    i = pl.program_id(0)
    hb = jnp.dot(x_ref[...], w_ref[...], preferred_element_type=jnp.float32)
    hx_ref[...] = hb
    axb = jnp.sum(hb * as_ref[...], axis=1, keepdims=True)
    adxb = jnp.sum(hb * ad_ref[...], axis=1, keepdims=True)
    ax_ref[...] = jnp.broadcast_to(axb, ax_ref.shape)
    adx_ref[...] = jnp.broadcast_to(adxb, adx_ref.shape)

    @pl.when(i == 0)
    def _():
        mx_sc[0, 0] = -jnp.inf
        mx_sc[0, 1] = -jnp.inf

    mx_sc[0, 0] = jnp.maximum(mx_sc[0, 0], jnp.max(axb))
    mx_sc[0, 1] = jnp.maximum(mx_sc[0, 1], jnp.max(adxb))

    @pl.when(i == pl.num_programs(0) - 1)
    def _():
        r = mx_sc[0, 0] + mx_sc[0, 1]
        m_ref[...] = jnp.where(r > 0.0, r, 0.2 * r).reshape(1, 1)


def _k1(x, W, att_src, att_dst):
    BN = 1000
    return pl.pallas_call(
        _k1_body,
        grid=(N // BN,),
        in_specs=[
            pl.BlockSpec((BN, D), lambda i: (i, 0)),
            pl.BlockSpec((D, D), lambda i: (0, 0)),
            pl.BlockSpec((1, D), lambda i: (0, 0)),
            pl.BlockSpec((1, D), lambda i: (0, 0)),
        ],
        out_specs=[
            pl.BlockSpec((BN, D), lambda i: (i, 0)),
            pl.BlockSpec((BN, 16), lambda i: (i, 0)),
            pl.BlockSpec((BN, 16), lambda i: (i, 0)),
            pl.BlockSpec((1, 1), lambda i: (0, 0)),
        ],
        out_shape=[
            jax.ShapeDtypeStruct((N, D), jnp.float32),
            jax.ShapeDtypeStruct((N, 16), jnp.float32),
            jax.ShapeDtypeStruct((N, 16), jnp.float32),
            jax.ShapeDtypeStruct((1, 1), jnp.float32),
        ],
        scratch_shapes=[pltpu.SMEM((1, 2), jnp.float32)],
    )(x, W, att_src.reshape(1, D), att_dst.reshape(1, D))


def _k2_body(hx_h, ax_h, adx_h, perm_h, src_h, dst_h, m_h,
             acc_h, den_h, hxp0_h, hxp1_h, axp0_h, axp1_h, adxp0_h, adxp1_h,
             src_v, p_v, m_v, src_all, dst_all,
             dsts_a, p_mat_a, rows_a, avb_a, bvb_a,
             dsts_b, p_mat_b, rows_b, avb_b, bvb_b,
             sem, sga, sgb, ssa, ssb, out_sh, den_sh):
    cid = lax.axis_index("c")
    sid = lax.axis_index("s")
    wid = sid * NC + cid
    nk = jnp.where(sid == NS - 1, LASTR // CHUNK, RPT // CHUNK)

    pltpu.sync_copy(m_h, m_v)

    zero16 = jnp.zeros((16,), jnp.float32)

    def _zero_rows(r, c):
        for j in range(D // 16):
            rows_a[r, pl.ds(j * 16, 16)] = zero16
        return c
    lax.fori_loop(0, CHUNK, _zero_rows, 0)

    def _zero_pm(r, c):
        p_mat_a[r, pl.ds(0, 16)] = zero16
        return c
    lax.fori_loop(0, CHUNK, _zero_pm, 0)

    def _zero_stripes(k, c):
        off = pl.ds(sid * RPT + k * CHUNK, CHUNK)
        pltpu.sync_copy(rows_a, out_sh.at[off])
        pltpu.sync_copy(p_mat_a, den_sh.at[off])
        return c
    lax.fori_loop(0, nk, _zero_stripes, 0)

    def _build(k, c):
        off = sid * RPT + k * CHUNK
        pltpu.sync_copy(perm_h.at[pl.ds(off, CHUNK)], src_v)
        g1 = pltpu.async_copy(hx_h.at[src_v], rows_a, sem)
        g2 = pltpu.async_copy(ax_h.at[src_v], avb_a, sem)
        g3 = pltpu.async_copy(adx_h.at[src_v], bvb_a, sem)
        g1.wait()
        g2.wait()
        g3.wait()

        @pl.when(cid == 0)
        def _():
            pltpu.sync_copy(rows_a, hxp0_h.at[pl.ds(off, CHUNK)])
            pltpu.sync_copy(avb_a, axp0_h.at[pl.ds(off, CHUNK)])
            pltpu.sync_copy(bvb_a, adxp0_h.at[pl.ds(off, CHUNK)])

        @pl.when(cid == 1)
        def _():
            pltpu.sync_copy(rows_a, hxp1_h.at[pl.ds(off, CHUNK)])
            pltpu.sync_copy(avb_a, axp1_h.at[pl.ds(off, CHUNK)])
            pltpu.sync_copy(bvb_a, adxp1_h.at[pl.ds(off, CHUNK)])
        return c
    lax.fori_loop(0, nk, _build, 0)

    ca = pltpu.async_copy(src_h.at[pl.ds(wid * EPW, EPW)], src_all, sem)
    cb = pltpu.async_copy(dst_h.at[pl.ds(wid * EPW, EPW)], dst_all, sem)
    ca.wait()
    cb.wait()

    plsc.subcore_barrier()

    mv = m_v[...]
    ziota = lax.iota(jnp.int32, 16) * 0

    def issue_g(i, dsts, rows, avb, bvb, sg):
        sl_s = src_all.at[pl.ds(i * CHUNK, CHUNK)]
        sl_d = dst_all.at[pl.ds(i * CHUNK, CHUNK)]
        pltpu.async_copy(dst_h.at[pl.ds(wid * EPW + i * CHUNK, CHUNK)],
                         dsts, sg)

        @pl.when(cid == 0)
        def _():
            pltpu.async_copy(axp0_h.at[sl_s], avb, sg)
            pltpu.async_copy(adxp0_h.at[sl_d], bvb, sg)
            pltpu.async_copy(hxp0_h.at[sl_s], rows, sg)

        @pl.when(cid == 1)
        def _():
            pltpu.async_copy(axp1_h.at[sl_s], avb, sg)
            pltpu.async_copy(adxp1_h.at[sl_d], bvb, sg)
            pltpu.async_copy(hxp1_h.at[sl_s], rows, sg)

    def drain_g(i, dsts, rows, avb, bvb, sg):
        sl_s = src_all.at[pl.ds(i * CHUNK, CHUNK)]
        sl_d = dst_all.at[pl.ds(i * CHUNK, CHUNK)]
        pltpu.make_async_copy(
            dst_h.at[pl.ds(wid * EPW + i * CHUNK, CHUNK)], dsts, sg).wait()
        pltpu.make_async_copy(axp0_h.at[sl_s], avb, sg).wait()
        pltpu.make_async_copy(adxp0_h.at[sl_d], bvb, sg).wait()
        pltpu.make_async_copy(hxp0_h.at[sl_s], rows, sg).wait()

    def compute(rows, avb, bvb, p_mat):
        for j in range(CHUNK // 16):
            sl = pl.ds(j * 16, 16)
            ridx = lax.iota(jnp.int32, 16) + j * 16
            av = plsc.load_gather(avb, [ridx, ziota])
            bv = plsc.load_gather(bvb, [ridx, ziota])
            raw = av + bv
            e = jnp.where(raw > 0.0, raw, raw * 0.2)
            p_v[sl] = jnp.exp(e - mv)

        def _scale(r, cc):
            pr = plsc.load_gather(p_v, [jnp.zeros((16,), jnp.int32) + r])
            for j in range(D // 16):
                csl = pl.ds(j * 16, 16)
                rows[r, csl] = rows[r, csl] * pr
            p_mat[r, pl.ds(0, 16)] = pr
            return cc
        lax.fori_loop(0, CHUNK, _scale, 0)

    def issue_s(dsts, rows, p_mat, ss):
        pltpu.async_copy(rows, out_sh.at[dsts], ss, add=True)
        pltpu.async_copy(p_mat, den_sh.at[dsts], ss, add=True)

    def drain_s(dsts, rows, p_mat, ss):
        pltpu.make_async_copy(rows, out_sh.at[dsts], ss).wait()
        pltpu.make_async_copy(p_mat, den_sh.at[dsts], ss).wait()

    seta = (dsts_a, rows_a, avb_a, bvb_a)
    setb = (dsts_b, rows_b, avb_b, bvb_b)

    issue_g(0, dsts_a, rows_a, avb_a, bvb_a, sga)

    def _pair(k, c):
        i0 = 2 * k
        i1 = 2 * k + 1

        @pl.when(k > 0)
        def _():
            drain_s(dsts_b, rows_b, p_mat_b, ssb)
        issue_g(i1, dsts_b, rows_b, avb_b, bvb_b, sgb)
        drain_g(i0, dsts_a, rows_a, avb_a, bvb_a, sga)
        compute(rows_a, avb_a, bvb_a, p_mat_a)
        issue_s(dsts_a, rows_a, p_mat_a, ssa)
        drain_g(i1, dsts_b, rows_b, avb_b, bvb_b, sgb)
        compute(rows_b, avb_b, bvb_b, p_mat_b)
        drain_s(dsts_a, rows_a, p_mat_a, ssa)

        @pl.when(i0 + 2 < NCHUNK)
        def _():
            issue_g(i0 + 2, dsts_a, rows_a, avb_a, bvb_a, sga)
        issue_s(dsts_b, rows_b, p_mat_b, ssb)
        return c

    lax.fori_loop(0, NCHUNK // 2, _pair, 0)

    drain_s(dsts_b, rows_b, p_mat_b, ssb)

    plsc.subcore_barrier()

    @pl.when(sid < NS - 1)
    def _():
        pltpu.sync_copy(out_sh.at[pl.ds(sid * RPT, RPT)],
                        acc_h.at[pl.ds(cid * N + sid * RPT, RPT)])
        pltpu.sync_copy(den_sh.at[pl.ds(sid * RPT, RPT)],
                        den_h.at[pl.ds(cid * N + sid * RPT, RPT)])

    @pl.when(sid == NS - 1)
    def _():
        pltpu.sync_copy(out_sh.at[pl.ds((NS - 1) * RPT, LASTR)],
                        acc_h.at[pl.ds(cid * N + (NS - 1) * RPT, LASTR)])
        pltpu.sync_copy(den_sh.at[pl.ds((NS - 1) * RPT, LASTR)],
                        den_h.at[pl.ds(cid * N + (NS - 1) * RPT, LASTR)])


def _k2(hx, ax16, adx16, perm_i, src, dst, mvec):
    mesh = plsc.VectorSubcoreMesh(core_axis_name="c", subcore_axis_name="s")
    f = pl.kernel(
        _k2_body,
        out_type=[
            jax.ShapeDtypeStruct((NC * N, D), jnp.float32),
            jax.ShapeDtypeStruct((NC * N, 16), jnp.float32),
            jax.ShapeDtypeStruct((N, D), jnp.float32),
            jax.ShapeDtypeStruct((N, D), jnp.float32),
            jax.ShapeDtypeStruct((N, 16), jnp.float32),
            jax.ShapeDtypeStruct((N, 16), jnp.float32),
            jax.ShapeDtypeStruct((N, 16), jnp.float32),
            jax.ShapeDtypeStruct((N, 16), jnp.float32),
        ],
        mesh=mesh,
        compiler_params=pltpu.CompilerParams(needs_layout_passes=False,
                                            use_tc_tiling_on_sc=False),
        scratch_types=[
            pltpu.VMEM((CHUNK,), jnp.int32),
            pltpu.VMEM((CHUNK,), jnp.float32),
            pltpu.VMEM((16,), jnp.float32),
            pltpu.VMEM((EPW,), jnp.int32),
            pltpu.VMEM((EPW,), jnp.int32),
            pltpu.VMEM((CHUNK,), jnp.int32),
            pltpu.VMEM((CHUNK, 16), jnp.float32),
            pltpu.VMEM((CHUNK, D), jnp.float32),
            pltpu.VMEM((CHUNK, 16), jnp.float32),
            pltpu.VMEM((CHUNK, 16), jnp.float32),
            pltpu.VMEM((CHUNK,), jnp.int32),
            pltpu.VMEM((CHUNK, 16), jnp.float32),
            pltpu.VMEM((CHUNK, D), jnp.float32),
            pltpu.VMEM((CHUNK, 16), jnp.float32),
            pltpu.VMEM((CHUNK, 16), jnp.float32),
            pltpu.SemaphoreType.DMA,
            pltpu.SemaphoreType.DMA,
            pltpu.SemaphoreType.DMA,
            pltpu.SemaphoreType.DMA,
            pltpu.SemaphoreType.DMA,
            pltpu.VMEM_SHARED((N, D), jnp.float32),
            pltpu.VMEM_SHARED((N, 16), jnp.float32),
        ],
    )
    return f(hx, ax16, adx16, perm_i, src, dst, mvec)


def _k3_body(a0_ref, a1_ref, d0_ref, d1_ref, wh_ref, bh_ref, out_ref,
             acc_sc):
    i = pl.program_id(0)
    dsum = (d0_ref[...] + d1_ref[...])[:, 0]
    a = a0_ref[...] + a1_ref[...]
    o = a / (dsum[:, None] + 1e-16)
    o = jnp.where(o > 0.0, o, jnp.exp(jnp.minimum(o, 0.0)) - 1.0)

    @pl.when(i == 0)
    def _():
        acc_sc[...] = jnp.zeros_like(acc_sc)

    acc_sc[...] += jnp.sum(o, axis=0, keepdims=True)

    @pl.when(i == pl.num_programs(0) - 1)
    def _():
        out_ref[...] = (jnp.sum(acc_sc[...] * wh_ref[...].T) / N
                        + jnp.sum(bh_ref[...])).reshape(1, 1)


def _k3(acc0, acc1, den0, den1, W_head, b_head):
    BN = 1000
    return pl.pallas_call(
        _k3_body,
        grid=(N // BN,),
        in_specs=[
            pl.BlockSpec((BN, D), lambda i: (i, 0)),
            pl.BlockSpec((BN, D), lambda i: (i, 0)),
            pl.BlockSpec((BN, 16), lambda i: (i, 0)),
            pl.BlockSpec((BN, 16), lambda i: (i, 0)),
            pl.BlockSpec((D, 1), lambda i: (0, 0)),
            pl.BlockSpec((1, 1), lambda i: (0, 0)),
        ],
        out_specs=pl.BlockSpec((1, 1), lambda i: (0, 0)),
        out_shape=jax.ShapeDtypeStruct((1, 1), jnp.float32),
        scratch_shapes=[pltpu.VMEM((1, D), jnp.float32)],
    )(acc0, acc1, den0, den1, W_head, b_head.reshape(1, 1))


def kernel(x, perm, edge_index, W, att_src, att_dst, W_head, b_head):
    hx, ax16, adx16, M = _k1(x, W, att_src, att_dst)

    perm_i = perm.astype(jnp.int32)
    src = edge_index[0].astype(jnp.int32)
    dst = edge_index[1].astype(jnp.int32)
    mvec = jnp.broadcast_to(M.reshape(1), (16,))

    acc, den = _k2(hx, ax16, adx16, perm_i, src, dst, mvec)[:2]

    pred = _k3(acc[:N], acc[N:], den[:N], den[N:], W_head, b_head)
    return pred.reshape(1)
